# baseline (speedup 1.0000x reference)
import jax
import jax.numpy as jnp
from jax import lax
from jax.experimental import pallas as pl

_L = 3
_ALPHA = 0.9
_EPS = 1e-08


def _dg(a, b, ca, cb):
    return lax.dot_general(
        a, b, (((ca,), (cb,)), ((), ())),
        preferred_element_type=jnp.float32)


def _one_batch(X, adj, Ft, MH, mn, memw, base_dm):
    C, N = X.shape
    A = Ft.shape[1]
    inv2 = 1.0 / (2.0 + _EPS)

    xn = X * lax.rsqrt(jnp.maximum(jnp.sum(X * X, axis=0, keepdims=True),
                                   1e-24))
    sim = _dg(xn, mn, 0, 0)

    ids_a = lax.broadcasted_iota(jnp.int32, (N, A), 1)
    ids_n = lax.broadcasted_iota(jnp.int32, (N, A), 0)
    big = jnp.int32(2 ** 30)
    mx_a = jnp.max(sim, axis=1, keepdims=True)
    knn_col = jnp.min(jnp.where(sim >= mx_a, ids_a, big), axis=1,
                      keepdims=True)
    knn_oh = (ids_a == knn_col).astype(jnp.float32)
    mx_n = jnp.max(sim, axis=0, keepdims=True)
    rev_row = jnp.min(jnp.where(sim >= mx_n, ids_n, big), axis=0,
                      keepdims=True)
    rev_oh = (ids_n == rev_row).astype(jnp.float32)

    M = (knn_oh + rev_oh) * inv2
    ones_a = jnp.ones((1, A), jnp.float32)
    r = _dg(ones_a, rev_oh, 1, 1)
    c = jnp.sum(knn_oh, axis=0, keepdims=True)

    adjw = adj / (jnp.sum(adj, axis=0, keepdims=True) + _EPS)
    dgq = (1.0 + r) * inv2 + 1.0
    dgm = (1.0 + c) * inv2 + 1.0

    Dq = (_dg(jnp.sum(adjw, axis=0, keepdims=True), adj, 1, 1)
          + 2.0 * dgq - 1.0 + _EPS)
    Dm = base_dm + 2.0 * dgm - 1.0 + _EPS
    inv_dq, inv_dm = 1.0 / Dq, 1.0 / Dm
    dis_q = lax.rsqrt(Dq)

    Zq = X * jnp.sqrt(Dq)
    Zm = Ft * jnp.sqrt(Dm)
    coef = [_ALPHA]
    for l in range(1, _L):
        coef.append(_ALPHA * (1.0 - _ALPHA) ** l)
    coef.append((1.0 - _ALPHA) ** _L)
    acc = coef[0] * Zq
    for l in range(1, _L + 1):
        Uq = Zq * inv_dq
        Um = Zm * inv_dm
        Zq = (_dg(_dg(Uq, adjw, 1, 0), adj, 1, 1) + Uq * dgq
              + _dg(Um, M, 1, 1))
        if l < _L:
            Zm = (_dg(_dg(Um, memw, 1, 0), MH, 1, 1) + Um * dgm
                  + _dg(Uq, M, 1, 0))
        acc = acc + coef[l] * Zq
    Xh = acc * dis_q

    e_inv = 1.0 / jnp.maximum(jnp.sum(adj, axis=0, keepdims=True), _EPS)
    v_isq = lax.rsqrt(jnp.maximum(jnp.sum(adj, axis=1, keepdims=True), _EPS))
    adj_s = adj * e_inv * v_isq
    xe = _dg(adj_s, Xh, 0, 1)
    return Xh, xe


def _bitfmp_kernel(f_ref, adj_ref, ftr_ref, mh_ref, xhat_ref, xe_ref):
    Ft = ftr_ref[...]
    MH = mh_ref[...]
    mn = Ft * lax.rsqrt(jnp.maximum(jnp.sum(Ft * Ft, axis=0, keepdims=True),
                                    1e-24))
    memw = MH / (jnp.sum(MH, axis=0, keepdims=True) + _EPS)
    base_dm = _dg(jnp.sum(memw, axis=0, keepdims=True), MH, 1, 1)
    for b in range(f_ref.shape[0]):
        Xh, xe = _one_batch(f_ref[b], adj_ref[b], Ft, MH, mn, memw, base_dm)
        xhat_ref[b] = Xh
        xe_ref[b] = xe


@jax.jit
def kernel(features, adj, total_feats, mem_adj):
    B, C, N = features.shape
    A, K = mem_adj.shape
    ft_t = total_feats.T.astype(jnp.float32)
    out = pl.pallas_call(
        _bitfmp_kernel,
        out_shape=[
            jax.ShapeDtypeStruct((B, C, N), jnp.float32),
            jax.ShapeDtypeStruct((B, K, C), jnp.float32),
        ],
    )(features.astype(jnp.float32), adj.astype(jnp.float32), ft_t,
      mem_adj.astype(jnp.float32))
    return tuple(out)


# speedup vs baseline: 1.0862x; 1.0862x over previous
import jax
import jax.numpy as jnp
from jax import lax
from jax.experimental import pallas as pl

_L = 3
_ALPHA = 0.9
_EPS = 1e-08


def _dg(a, b, ca, cb):
    return lax.dot_general(
        a, b, (((ca,), (cb,)), ((), ())),
        preferred_element_type=jnp.float32)


def _one_batch(X, adj, Ft, MH, mn, memw, base_dm):
    C, N = X.shape
    A = Ft.shape[1]
    inv2 = 1.0 / (2.0 + _EPS)

    xn = X * lax.rsqrt(jnp.maximum(jnp.sum(X * X, axis=0, keepdims=True),
                                   1e-24))
    sim = _dg(xn, mn, 0, 0)

    ids_a = lax.broadcasted_iota(jnp.int32, (N, A), 1)
    ids_n = lax.broadcasted_iota(jnp.int32, (N, A), 0)
    big = jnp.int32(2 ** 30)
    mx_a = jnp.max(sim, axis=1, keepdims=True)
    knn_col = jnp.min(jnp.where(sim >= mx_a, ids_a, big), axis=1,
                      keepdims=True)
    knn_oh = (ids_a == knn_col).astype(jnp.float32)
    mx_n = jnp.max(sim, axis=0, keepdims=True)
    rev_row = jnp.min(jnp.where(sim >= mx_n, ids_n, big), axis=0,
                      keepdims=True)
    rev_oh = (ids_n == rev_row).astype(jnp.float32)

    M = (knn_oh + rev_oh) * inv2
    ones_a = jnp.ones((1, A), jnp.float32)
    r = _dg(ones_a, rev_oh, 1, 1)
    c = jnp.sum(knn_oh, axis=0, keepdims=True)

    adjw = adj / (jnp.sum(adj, axis=0, keepdims=True) + _EPS)
    dgq = (1.0 + r) * inv2 + 1.0
    dgm = (1.0 + c) * inv2 + 1.0

    Dq = (_dg(jnp.sum(adjw, axis=0, keepdims=True), adj, 1, 1)
          + 2.0 * dgq - 1.0 + _EPS)
    Dm = base_dm + 2.0 * dgm - 1.0 + _EPS
    inv_dq, inv_dm = 1.0 / Dq, 1.0 / Dm
    dis_q = lax.rsqrt(Dq)

    Zq = X * jnp.sqrt(Dq)
    Zm = Ft * jnp.sqrt(Dm)
    coef = [_ALPHA]
    for l in range(1, _L):
        coef.append(_ALPHA * (1.0 - _ALPHA) ** l)
    coef.append((1.0 - _ALPHA) ** _L)
    acc = coef[0] * Zq
    for l in range(1, _L + 1):
        Uq = Zq * inv_dq
        Um = Zm * inv_dm
        Zq = (_dg(_dg(Uq, adjw, 1, 0), adj, 1, 1) + Uq * dgq
              + _dg(Um, M, 1, 1))
        if l < _L:
            Zm = (_dg(_dg(Um, memw, 1, 0), MH, 1, 1) + Um * dgm
                  + _dg(Uq, M, 1, 0))
        acc = acc + coef[l] * Zq
    Xh = acc * dis_q

    e_inv = 1.0 / jnp.maximum(jnp.sum(adj, axis=0, keepdims=True), _EPS)
    v_isq = lax.rsqrt(jnp.maximum(jnp.sum(adj, axis=1, keepdims=True), _EPS))
    adj_s = adj * e_inv * v_isq
    xe = _dg(adj_s, Xh, 0, 1)
    return Xh, xe


def _bitfmp_kernel(f_ref, adj_ref, ftr_ref, mh_ref, xhat_ref, xe_ref):
    Ft = jnp.transpose(ftr_ref[...])
    MH = mh_ref[...]
    mn = Ft * lax.rsqrt(jnp.maximum(jnp.sum(Ft * Ft, axis=0, keepdims=True),
                                    1e-24))
    memw = MH / (jnp.sum(MH, axis=0, keepdims=True) + _EPS)
    base_dm = _dg(jnp.sum(memw, axis=0, keepdims=True), MH, 1, 1)
    for b in range(f_ref.shape[0]):
        Xh, xe = _one_batch(f_ref[b], adj_ref[b], Ft, MH, mn, memw, base_dm)
        xhat_ref[b] = Xh
        xe_ref[b] = xe


@jax.jit
def kernel(features, adj, total_feats, mem_adj):
    B, C, N = features.shape
    A, K = mem_adj.shape
    out = pl.pallas_call(
        _bitfmp_kernel,
        out_shape=[
            jax.ShapeDtypeStruct((B, C, N), jnp.float32),
            jax.ShapeDtypeStruct((B, K, C), jnp.float32),
        ],
    )(features.astype(jnp.float32), adj.astype(jnp.float32),
      total_feats.astype(jnp.float32), mem_adj.astype(jnp.float32))
    return tuple(out)
